# Initial kernel scaffold; baseline (speedup 1.0000x reference)
import jax
import jax.numpy as jnp
from jax.experimental import pallas as pl


def kernel(x, codebooks):
    raise NotImplementedError("write your pallas kernel here")


# speedup vs baseline: 1.3266x; 1.3266x over previous
import functools

import jax
import jax.numpy as jnp
from jax import lax
from jax.experimental import pallas as pl
from jax.experimental.pallas import tpu as pltpu
from jax.experimental.pallas import tpu_sc as plsc

NUM_Q = 4
NB = 8192
D = 256
HALF = 128
N = 8192
BETA = 0.25
ENT_RATIO = 0.1
INV_TEMP = 100.0

RB = 512
NBLK = N // RB
CT = 1024
NCT = NB // CT
CBR = 256


def _norm_body(cb_ref, en_ref, esq_ref):
    c = cb_ref[0]
    c1 = c[:, :HALF]
    c2 = c[:, HALF:]
    s1 = jnp.sum(c1 * c1, axis=1, keepdims=True)
    s2 = jnp.sum(c2 * c2, axis=1, keepdims=True)
    en = jnp.concatenate(
        [c1 * lax.rsqrt(s1 + 1e-12), c2 * lax.rsqrt(s2 + 1e-12)], axis=1)
    en_ref[0] = en
    esq_ref[0, 0, :] = jnp.sum(en * en, axis=1)


def _normalize_codebooks(codebooks):
    return pl.pallas_call(
        _norm_body,
        grid=(NUM_Q, NB // CBR),
        in_specs=[pl.BlockSpec((1, CBR, D), lambda q, j: (q, j, 0))],
        out_specs=[
            pl.BlockSpec((1, CBR, D), lambda q, j: (q, j, 0)),
            pl.BlockSpec((1, 1, CBR), lambda q, j: (q, 0, j)),
        ],
        out_shape=[
            jax.ShapeDtypeStruct((NUM_Q, NB, D), jnp.float32),
            jax.ShapeDtypeStruct((NUM_Q, 1, NB), jnp.float32),
        ],
    )(codebooks)


def _stats_body(resid_ref, zqp_ref, enT_ref, esq_ref,
                resid_out_ref, idx_ref, a_ref, part_ref, used_ref, s_ref):
    blk = pl.program_id(0)
    r = resid_ref[...] - zqp_ref[...]
    resid_out_ref[...] = r
    r1 = r[:, :HALF]
    r2 = r[:, HALF:]
    s1 = jnp.sum(r1 * r1, axis=1, keepdims=True)
    s2 = jnp.sum(r2 * r2, axis=1, keepdims=True)
    zn = jnp.concatenate(
        [r1 * lax.rsqrt(s1 + 1e-12), r2 * lax.rsqrt(s2 + 1e-12)], axis=1)
    zsq = jnp.sum(zn * zn, axis=1, keepdims=True)

    def p1(j, m):
        st = j * CT
        s_t = jnp.dot(zn, enT_ref[:, pl.ds(st, CT)],
                      preferred_element_type=jnp.float32)
        y_t = 2.0 * s_t - esq_ref[:, pl.ds(st, CT)]
        s_ref[:, pl.ds(st, CT)] = y_t
        return jnp.maximum(m, jnp.max(y_t, axis=1, keepdims=True))

    m = lax.fori_loop(0, NCT, p1, jnp.full((RB, 1), -jnp.inf, jnp.float32))

    def p2(j, carry):
        zacc, tacc, iacc = carry
        st = j * CT
        y_t = s_ref[:, pl.ds(st, CT)]
        g = (y_t - m) * INV_TEMP
        e = jnp.exp(g)
        s_ref[:, pl.ds(st, CT)] = e
        zacc = zacc + jnp.sum(e, axis=1, keepdims=True)
        tacc = tacc + jnp.sum(e * g, axis=1, keepdims=True)
        ii = lax.broadcasted_iota(jnp.int32, (RB, CT), 1) + st
        cand = jnp.min(jnp.where(y_t >= m, ii, NB), axis=1, keepdims=True)
        return zacc, tacc, jnp.minimum(iacc, cand)

    zero = jnp.zeros((RB, 1), jnp.float32)
    z_s, t_s, idx = lax.fori_loop(
        0, NCT, p2, (zero, zero, jnp.full((RB, 1), NB, jnp.int32)))

    idx_ref[0, 0, :] = idx[:, 0]
    sent = jnp.log(z_s) - t_s / z_s
    lane = lax.broadcasted_iota(jnp.int32, (1, 1, 128), 2)
    part_ref[...] = (
        jnp.where(lane == 0, jnp.sum(sent), 0.0)
        + jnp.where(lane == 1, jnp.sum(zsq - m), 0.0)
        + jnp.where(lane == 2, jnp.sum(jnp.sqrt(s1)), 0.0)
        + jnp.where(lane == 3, jnp.sum(jnp.sqrt(s2)), 0.0))

    @pl.when(blk == 0)
    def _():
        a_ref[...] = jnp.zeros_like(a_ref)
        used_ref[...] = jnp.zeros_like(used_ref)

    inv_z = 1.0 / z_s

    def p3(j, _):
        st = j * CT
        e = s_ref[:, pl.ds(st, CT)]
        a_ref[:, pl.ds(st, CT)] += jnp.sum(e * inv_z, axis=0, keepdims=True)
        ii = lax.broadcasted_iota(jnp.int32, (RB, CT), 1) + st
        hit = jnp.max(jnp.where(idx == ii, 1.0, 0.0), axis=0, keepdims=True)
        used_ref[:, pl.ds(st, CT)] = jnp.maximum(
            used_ref[:, pl.ds(st, CT)], hit)
        return 0

    lax.fori_loop(0, NCT, p3, 0)


def _stats(resid, zq_prev, enT_q, esq_q):
    return pl.pallas_call(
        _stats_body,
        grid=(NBLK,),
        in_specs=[
            pl.BlockSpec((RB, D), lambda i: (i, 0)),
            pl.BlockSpec((RB, D), lambda i: (i, 0)),
            pl.BlockSpec((D, NB), lambda i: (0, 0)),
            pl.BlockSpec((1, NB), lambda i: (0, 0)),
        ],
        out_specs=[
            pl.BlockSpec((RB, D), lambda i: (i, 0)),
            pl.BlockSpec((1, 1, RB), lambda i: (i, 0, 0)),
            pl.BlockSpec((1, NB), lambda i: (0, 0)),
            pl.BlockSpec((1, 1, 128), lambda i: (i, 0, 0)),
            pl.BlockSpec((1, NB), lambda i: (0, 0)),
        ],
        out_shape=[
            jax.ShapeDtypeStruct((N, D), jnp.float32),
            jax.ShapeDtypeStruct((NBLK, 1, RB), jnp.int32),
            jax.ShapeDtypeStruct((1, NB), jnp.float32),
            jax.ShapeDtypeStruct((NBLK, 1, 128), jnp.float32),
            jax.ShapeDtypeStruct((1, NB), jnp.float32),
        ],
        scratch_shapes=[pltpu.VMEM((RB, NB), jnp.float32)],
    )(resid, zq_prev, enT_q, esq_q)


_SC_NW = 32
_BPW = N // _SC_NW


def _sc_gather_body(en_hbm, idx_hbm, zq_hbm, idx_v, rows_v, sem):
    wid = lax.axis_index("s") * 2 + lax.axis_index("c")
    pltpu.sync_copy(idx_hbm.at[pl.ds(wid * 2, 2)], idx_v)
    cp0 = pltpu.async_copy(en_hbm.at[idx_v.at[0]],
                           rows_v.at[pl.ds(0, 128)], sem)
    cp1 = pltpu.async_copy(en_hbm.at[idx_v.at[1]],
                           rows_v.at[pl.ds(128, 128)], sem)
    cp0.wait()
    cp1.wait()
    pltpu.sync_copy(rows_v, zq_hbm.at[pl.ds(wid * _BPW, _BPW)])


def _sc_gather(en_q, idx_flat):
    idx2 = idx_flat.reshape(_SC_NW * 2, 128)
    mesh = plsc.VectorSubcoreMesh(core_axis_name="c", subcore_axis_name="s")
    f = functools.partial(
        pl.kernel,
        mesh=mesh,
        out_type=jax.ShapeDtypeStruct((N, D), jnp.float32),
        scratch_types=[
            pltpu.VMEM((2, 128), jnp.int32),
            pltpu.VMEM((_BPW, D), jnp.float32),
            pltpu.SemaphoreType.DMA,
        ],
    )(_sc_gather_body)
    return f(en_q, idx2)


def _final_body(a_ref, parts_ref, used_ref, out_ref):
    parts = parts_ref[...]
    tot = [0.0] * 6
    for q in range(NUM_Q):
        p = parts[q * NBLK:(q + 1) * NBLK, :]
        sent_mean = jnp.sum(p[:, 0]) / N
        av = a_ref[q:q + 1, :] * (1.0 / N)
        avg_ent = -jnp.sum(av * jnp.log(av + 1e-5))
        u = used_ref[q:q + 1, :]
        usage = jnp.sum((u > 0).astype(jnp.float32)) * (1.0 / NB)
        vq = jnp.sum(p[:, 1]) * (1.0 / (N * D))
        tot[0] += vq
        tot[1] += BETA * vq
        tot[2] += ENT_RATIO * (sent_mean - avg_ent)
        tot[3] += usage
        tot[4] += jnp.sum(p[:, 2]) / N
        tot[5] += jnp.sum(p[:, 3]) / N
    lane = lax.broadcasted_iota(jnp.int32, (1, 128), 1)
    acc = jnp.zeros((1, 128), jnp.float32)
    for i in range(6):
        acc = acc + jnp.where(lane == i, tot[i] / NUM_Q, 0.0)
    out_ref[...] = acc


def _finalize(a_all, parts_all, used_all):
    return pl.pallas_call(
        _final_body,
        grid=(1,),
        in_specs=[
            pl.BlockSpec((NUM_Q, NB), lambda i: (0, 0)),
            pl.BlockSpec((NUM_Q * NBLK, 128), lambda i: (0, 0)),
            pl.BlockSpec((NUM_Q, NB), lambda i: (0, 0)),
        ],
        out_specs=pl.BlockSpec((1, 128), lambda i: (0, 0)),
        out_shape=jax.ShapeDtypeStruct((1, 128), jnp.float32),
    )(a_all, parts_all, used_all)


def _qo_body(z_ref, r_ref, zq_ref, out_ref):
    out_ref[...] = z_ref[...] - r_ref[...] + zq_ref[...]


def _qo(z, resid_last, zq_last):
    return pl.pallas_call(
        _qo_body,
        grid=(NBLK,),
        in_specs=[pl.BlockSpec((RB, D), lambda i: (i, 0))] * 3,
        out_specs=pl.BlockSpec((RB, D), lambda i: (i, 0)),
        out_shape=jax.ShapeDtypeStruct((N, D), jnp.float32),
    )(z, resid_last, zq_last)


def kernel(x, codebooks):
    B, C, H, W = x.shape
    z = jnp.transpose(x, (0, 2, 3, 1)).reshape(-1, C)

    en, esq = _normalize_codebooks(codebooks)
    enT = jnp.swapaxes(en, 1, 2)

    resid = z
    zq_prev = jnp.zeros_like(z)
    idx_list, a_list, part_list, used_list = [], [], [], []
    for q in range(NUM_Q):
        resid, idxq, a_q, part_q, used_q = _stats(
            resid, zq_prev, enT[q], esq[q])
        idx_flat = idxq.reshape(N)
        zq_prev = _sc_gather(en[q], idx_flat)
        idx_list.append(idx_flat.reshape(B, H, W))
        a_list.append(a_q)
        part_list.append(part_q.reshape(NBLK, 128))
        used_list.append(used_q)

    qo_flat = _qo(z, resid, zq_prev)
    qo = jnp.transpose(qo_flat.reshape(B, H, W, C), (0, 3, 1, 2))
    all_indices = jnp.stack(idx_list, axis=-1)

    scal = _finalize(
        jnp.concatenate(a_list, axis=0),
        jnp.concatenate(part_list, axis=0),
        jnp.concatenate(used_list, axis=0),
    )
    return (qo, all_indices, scal[0, 0], scal[0, 1], scal[0, 2],
            scal[0, 3], scal[0, 4], scal[0, 5])
